# baseline (speedup 1.0000x reference)
import jax
import jax.numpy as jnp
from jax.experimental import pallas as pl
from jax.experimental.pallas import tpu as pltpu


def _copy_body(in_ref, out_ref, sem):
    copy = pltpu.make_async_copy(in_ref, out_ref, sem)
    copy.start()
    copy.wait()


def kernel(x, positional_embeddings):
    seq = x.shape[1]
    table = positional_embeddings
    src = table if seq == table.shape[0] else table[:seq]
    return pl.pallas_call(
        _copy_body,
        in_specs=[pl.BlockSpec(memory_space=pltpu.MemorySpace.HBM)],
        out_specs=pl.BlockSpec(memory_space=pltpu.MemorySpace.HBM),
        out_shape=jax.ShapeDtypeStruct((seq, table.shape[1]), table.dtype),
        scratch_shapes=[pltpu.SemaphoreType.DMA],
    )(src)


# speedup vs baseline: 1.0052x; 1.0052x over previous
import jax
import jax.numpy as jnp
from jax.experimental import pallas as pl
from jax.experimental.pallas import tpu as pltpu

_NUM_DMAS = 32


def _copy_body(in_ref, out_ref, sems):
    rows = in_ref.shape[0]
    chunk = rows // _NUM_DMAS
    copies = [
        pltpu.make_async_copy(
            in_ref.at[pl.ds(i * chunk, chunk)],
            out_ref.at[pl.ds(i * chunk, chunk)],
            sems.at[i],
        )
        for i in range(_NUM_DMAS)
    ]
    for c in copies:
        c.start()
    for c in copies:
        c.wait()


def kernel(x, positional_embeddings):
    seq = x.shape[1]
    table = positional_embeddings
    src = table if seq == table.shape[0] else table[:seq]
    return pl.pallas_call(
        _copy_body,
        in_specs=[pl.BlockSpec(memory_space=pltpu.MemorySpace.HBM)],
        out_specs=pl.BlockSpec(memory_space=pltpu.MemorySpace.HBM),
        out_shape=jax.ShapeDtypeStruct((seq, table.shape[1]), table.dtype),
        scratch_shapes=[pltpu.SemaphoreType.DMA((_NUM_DMAS,))],
    )(src)


# speedup vs baseline: 22.7869x; 22.6692x over previous
import functools

import jax
import jax.numpy as jnp
from jax import lax
from jax.experimental import pallas as pl
from jax.experimental.pallas import tpu as pltpu
from jax.experimental.pallas import tpu_sc as plsc

_NC = 2
_NS = 16
_NW = _NC * _NS
_CH = 32


def _make_sc_copy(seq, d, dtype):
    rows_per_w = seq // _NW
    nchunk = rows_per_w // _CH
    mesh = plsc.VectorSubcoreMesh(core_axis_name="c", subcore_axis_name="s")

    @functools.partial(
        pl.kernel,
        out_type=jax.ShapeDtypeStruct((seq, d), dtype),
        mesh=mesh,
        scratch_types=[
            pltpu.VMEM((2, _CH, d), dtype),
            pltpu.SemaphoreType.DMA((2,)),
            pltpu.SemaphoreType.DMA((2,)),
        ],
    )
    def sc_copy(table_hbm, out_hbm, buf, isem, osem):
        wid = lax.axis_index("s") * _NC + lax.axis_index("c")
        base = wid * rows_per_w

        def in_cp(i, s):
            return pltpu.make_async_copy(
                table_hbm.at[pl.ds(base + i * _CH, _CH)], buf.at[s], isem.at[s])

        def out_cp(i, s):
            return pltpu.make_async_copy(
                buf.at[s], out_hbm.at[pl.ds(base + i * _CH, _CH)], osem.at[s])

        in_cp(0, 0).start()
        for i in range(nchunk):
            s = i % 2
            in_cp(i, s).wait()
            out_cp(i, s).start()
            if i + 1 < nchunk:
                ns = (i + 1) % 2
                if i >= 1:
                    out_cp(i - 1, ns).wait()
                in_cp(i + 1, ns).start()
        out_cp(nchunk - 1, (nchunk - 1) % 2).wait()
        if nchunk >= 2:
            out_cp(nchunk - 2, (nchunk - 2) % 2).wait()

    return sc_copy


def kernel(x, positional_embeddings):
    seq = x.shape[1]
    table = positional_embeddings
    src = table if seq == table.shape[0] else table[:seq]
    return _make_sc_copy(seq, table.shape[1], table.dtype)(src)


# speedup vs baseline: 23.8589x; 1.0470x over previous
import functools

import jax
import jax.numpy as jnp
from jax import lax
from jax.experimental import pallas as pl
from jax.experimental.pallas import tpu as pltpu
from jax.experimental.pallas import tpu_sc as plsc

_NC = 2
_NS = 16
_NW = _NC * _NS
_CH = 16
_NBUF = 4


def _make_sc_copy(seq, d, dtype):
    rows_per_w = seq // _NW
    nchunk = rows_per_w // _CH
    mesh = plsc.VectorSubcoreMesh(core_axis_name="c", subcore_axis_name="s")

    @functools.partial(
        pl.kernel,
        out_type=jax.ShapeDtypeStruct((seq, d), dtype),
        mesh=mesh,
        scratch_types=[
            pltpu.VMEM((_NBUF, _CH, d), dtype),
            pltpu.SemaphoreType.DMA((_NBUF,)),
            pltpu.SemaphoreType.DMA((_NBUF,)),
        ],
    )
    def sc_copy(table_hbm, out_hbm, buf, isem, osem):
        wid = lax.axis_index("s") * _NC + lax.axis_index("c")
        base = wid * rows_per_w

        def in_cp(i):
            s = i % _NBUF
            return pltpu.make_async_copy(
                table_hbm.at[pl.ds(base + i * _CH, _CH)], buf.at[s], isem.at[s])

        def out_cp(i):
            s = i % _NBUF
            return pltpu.make_async_copy(
                buf.at[s], out_hbm.at[pl.ds(base + i * _CH, _CH)], osem.at[s])

        waited = set()
        for i in range(min(_NBUF - 1, nchunk)):
            in_cp(i).start()
        for i in range(nchunk):
            in_cp(i).wait()
            out_cp(i).start()
            nxt = i + _NBUF - 1
            if nxt < nchunk:
                if i >= 1:
                    out_cp(i - 1).wait()
                    waited.add(i - 1)
                in_cp(nxt).start()
        for j in range(nchunk):
            if j not in waited:
                out_cp(j).wait()

    return sc_copy


def kernel(x, positional_embeddings):
    seq = x.shape[1]
    table = positional_embeddings
    src = table if seq == table.shape[0] else table[:seq]
    return _make_sc_copy(seq, table.shape[1], table.dtype)(src)


# speedup vs baseline: 24.1581x; 1.0125x over previous
import functools

import jax
import jax.numpy as jnp
from jax import lax
from jax.experimental import pallas as pl
from jax.experimental.pallas import tpu as pltpu
from jax.experimental.pallas import tpu_sc as plsc

_NC = 2
_NS = 16
_NW = _NC * _NS
_CH = 32
_NBUF = 2


def _make_sc_copy(seq, d, dtype):
    rows_per_w = seq // _NW
    nchunk = rows_per_w // _CH
    mesh = plsc.VectorSubcoreMesh(core_axis_name="c", subcore_axis_name="s")

    @functools.partial(
        pl.kernel,
        out_type=jax.ShapeDtypeStruct((seq, d), dtype),
        mesh=mesh,
        scratch_types=[
            pltpu.VMEM_SHARED((_NS, _NBUF, _CH, d), dtype),
            pltpu.SemaphoreType.DMA((_NBUF,)),
            pltpu.SemaphoreType.DMA((_NBUF,)),
        ],
    )
    def sc_copy(table_hbm, out_hbm, buf, isem, osem):
        cid = lax.axis_index("c")
        sid = lax.axis_index("s")
        wid = sid * _NC + cid
        base = wid * rows_per_w

        def in_cp(i):
            s = i % _NBUF
            return pltpu.make_async_copy(
                table_hbm.at[pl.ds(base + i * _CH, _CH)],
                buf.at[sid, s], isem.at[s])

        def out_cp(i):
            s = i % _NBUF
            return pltpu.make_async_copy(
                buf.at[sid, s],
                out_hbm.at[pl.ds(base + i * _CH, _CH)], osem.at[s])

        waited = set()
        for i in range(min(_NBUF - 1, nchunk)):
            in_cp(i).start()
        for i in range(nchunk):
            in_cp(i).wait()
            out_cp(i).start()
            nxt = i + _NBUF - 1
            if nxt < nchunk:
                if i >= 1:
                    out_cp(i - 1).wait()
                    waited.add(i - 1)
                in_cp(nxt).start()
        for j in range(nchunk):
            if j not in waited:
                out_cp(j).wait()

    return sc_copy


def kernel(x, positional_embeddings):
    seq = x.shape[1]
    table = positional_embeddings
    src = table if seq == table.shape[0] else table[:seq]
    return _make_sc_copy(seq, table.shape[1], table.dtype)(src)
